# baseline (speedup 1.0000x reference)
import jax
import jax.numpy as jnp
from jax import lax
from jax.experimental import pallas as pl

S, E = 4096, 64
CAP = 80
NOISE_HI = 0x3F800000
TS = 256


def _col_cumsum_excl(x):
    y = x
    sh = 1
    while sh < S:
        y = y + jnp.pad(y, ((sh, 0), (0, 0)))[:S, :]
        sh *= 2
    return y - x


def _route_kernel(x_ref, n_ref, wd_ref, tgt_ref, cnt_ref):
    x = x_ref[...]
    noise = n_ref[...]
    col = lax.broadcasted_iota(jnp.int32, (S, E), 1)

    row_max = jnp.max(x, axis=1, keepdims=True)
    is_max = x >= row_max
    e_idx = jnp.min(jnp.where(is_max, col, E), axis=1, keepdims=True)
    emask = col == e_idx
    z = jnp.sum(jnp.exp(x - row_max), axis=1, keepdims=True)
    prob = 1.0 / z

    cnt_ref[...] = jnp.sum(emask.astype(jnp.int32), axis=0, keepdims=True)

    keys = lax.bitcast_convert_type(jnp.where(emask, noise, 0.0), jnp.int32)

    def body(_, carry):
        lo, hi = carry
        mid = lo + (hi - lo + 1) // 2
        cge = jnp.sum((keys >= mid).astype(jnp.int32), axis=0, keepdims=True)
        ge = cge >= CAP
        return jnp.where(ge, mid, lo), jnp.where(ge, hi, mid - 1)

    lo0 = jnp.zeros((1, E), jnp.int32)
    hi0 = jnp.full((1, E), NOISE_HI, jnp.int32)
    vcap, _ = lax.fori_loop(0, 31, body, (lo0, hi0))

    n_gt = jnp.sum((keys > vcap).astype(jnp.int32), axis=0, keepdims=True)
    ties_needed = CAP - n_gt
    is_tie = keys == vcap
    tie_rank = _col_cumsum_excl(is_tie.astype(jnp.int32))
    sel = (keys > vcap) | (is_tie & (tie_rank < ties_needed))
    disp = emask & sel

    loc_x = _col_cumsum_excl(disp.astype(jnp.int32))
    loc = jnp.sum(jnp.where(disp, loc_x, 0), axis=1, keepdims=True)
    disp_t = jnp.any(disp, axis=1, keepdims=True)

    wd_ref[...] = jnp.where(disp_t, prob, 0.0)
    tgt_ref[...] = jnp.where(disp_t, e_idx * CAP + loc, -1)


def _fill_kernel(wd_ref, tgt_ref, cw_ref, m_ref):
    j = lax.broadcasted_iota(jnp.int32, (TS, E * CAP), 1)
    hit = j == tgt_ref[...]
    cw_ref[...] = jnp.where(hit, wd_ref[...], 0.0)
    m_ref[...] = hit


def kernel(inputs, rand_noise):
    wd, tgt, cnt = pl.pallas_call(
        _route_kernel,
        out_shape=[
            jax.ShapeDtypeStruct((S, 1), jnp.float32),
            jax.ShapeDtypeStruct((S, 1), jnp.int32),
            jax.ShapeDtypeStruct((1, E), jnp.int32),
        ],
    )(inputs, rand_noise)
    cw, m = pl.pallas_call(
        _fill_kernel,
        grid=(S // TS,),
        in_specs=[
            pl.BlockSpec((TS, 1), lambda i: (i, 0)),
            pl.BlockSpec((TS, 1), lambda i: (i, 0)),
        ],
        out_specs=[
            pl.BlockSpec((TS, E * CAP), lambda i: (i, 0)),
            pl.BlockSpec((TS, E * CAP), lambda i: (i, 0)),
        ],
        out_shape=[
            jax.ShapeDtypeStruct((S, E * CAP), jnp.float32),
            jax.ShapeDtypeStruct((S, E * CAP), jnp.bool_),
        ],
    )(wd, tgt)
    return cw.reshape(S, E, CAP), m.reshape(S, E, CAP), cnt.reshape(E)
